# baseline (speedup 1.0000x reference)
import jax
import jax.numpy as jnp
from jax.experimental import pallas as pl
from jax.experimental.pallas import tpu as pltpu

_N_CHUNKS = 4


def _dma_pipeline(w_ref, o_ref, buf, in_sems, out_sems):
    seq_len = o_ref.shape[1]
    chunk = seq_len // _N_CHUNKS
    ins = [
        pltpu.make_async_copy(
            w_ref.at[0, pl.ds(i * chunk, chunk), :],
            buf.at[i],
            in_sems.at[i],
        )
        for i in range(_N_CHUNKS)
    ]
    outs = [
        pltpu.make_async_copy(
            buf.at[i],
            o_ref.at[0, pl.ds(i * chunk, chunk), :],
            out_sems.at[i],
        )
        for i in range(_N_CHUNKS)
    ]
    ins[0].start()
    for i in range(_N_CHUNKS):
        ins[i].wait()
        if i + 1 < _N_CHUNKS:
            ins[i + 1].start()
        outs[i].start()
    for c in outs:
        c.wait()


def kernel(x, weight):
    seq_len = x.shape[1]
    d_model = weight.shape[2]
    chunk = seq_len // _N_CHUNKS
    return pl.pallas_call(
        _dma_pipeline,
        in_specs=[pl.BlockSpec(memory_space=pl.ANY)],
        out_specs=pl.BlockSpec(memory_space=pl.ANY),
        out_shape=jax.ShapeDtypeStruct((1, seq_len, d_model), weight.dtype),
        scratch_shapes=[
            pltpu.VMEM((_N_CHUNKS, chunk, d_model), weight.dtype),
            pltpu.SemaphoreType.DMA((_N_CHUNKS,)),
            pltpu.SemaphoreType.DMA((_N_CHUNKS,)),
        ],
    )(weight)


# speedup vs baseline: 1.4290x; 1.4290x over previous
import jax
import jax.numpy as jnp
from jax.experimental import pallas as pl
from jax.experimental.pallas import tpu as pltpu

_N_CHUNKS = 2


def _dma_pipeline(w_ref, o_ref, buf, in_sems, out_sems):
    seq_len = o_ref.shape[1]
    chunk = seq_len // _N_CHUNKS
    ins = [
        pltpu.make_async_copy(
            w_ref.at[0, pl.ds(i * chunk, chunk), :],
            buf.at[i],
            in_sems.at[i],
        )
        for i in range(_N_CHUNKS)
    ]
    outs = [
        pltpu.make_async_copy(
            buf.at[i],
            o_ref.at[0, pl.ds(i * chunk, chunk), :],
            out_sems.at[i],
        )
        for i in range(_N_CHUNKS)
    ]
    for c in ins:
        c.start()
    for i in range(_N_CHUNKS):
        ins[i].wait()
        outs[i].start()
    for c in outs:
        c.wait()


def kernel(x, weight):
    seq_len = x.shape[1]
    d_model = weight.shape[2]
    chunk = seq_len // _N_CHUNKS
    return pl.pallas_call(
        _dma_pipeline,
        in_specs=[pl.BlockSpec(memory_space=pl.ANY)],
        out_specs=pl.BlockSpec(memory_space=pl.ANY),
        out_shape=jax.ShapeDtypeStruct((1, seq_len, d_model), weight.dtype),
        scratch_shapes=[
            pltpu.VMEM((_N_CHUNKS, chunk, d_model), weight.dtype),
            pltpu.SemaphoreType.DMA((_N_CHUNKS,)),
            pltpu.SemaphoreType.DMA((_N_CHUNKS,)),
        ],
    )(weight)
